# baseline (speedup 1.0000x reference)
import functools

import jax
import jax.numpy as jnp
from jax import lax
from jax.experimental import pallas as pl
from jax.experimental.pallas import tpu as pltpu
from jax.experimental.pallas import tpu_sc as plsc

_FEATURE = 1024
_QSIZE = 256
_NCLASS = 100
_BATCH = 4096

_NC = 2
_NS = 16
_L = 16
_NW = _NC * _NS
_CPAD = 128


def _norm_body(emb_ref, out_ref):
    x = emb_ref[...]
    n = jnp.sqrt(jnp.sum(x * x, axis=1, keepdims=True))
    out_ref[...] = x / jnp.maximum(n, 1e-12)


def _normalize(embeddings):
    blk = 512
    return pl.pallas_call(
        _norm_body,
        grid=(_BATCH // blk,),
        in_specs=[pl.BlockSpec((blk, _FEATURE), lambda i: (i, 0))],
        out_specs=pl.BlockSpec((blk, _FEATURE), lambda i: (i, 0)),
        out_shape=jax.ShapeDtypeStruct((_BATCH, _FEATURE), jnp.float32),
    )(embeddings)


def _sc_body(labels_hbm, ptr_hbm, queue_hbm, emb_hbm,
             out_hbm, newptr_hbm,
             labels_v, ptr_v, match_v, rows_v, ptrbuf_v,
             copy_sem, row_sem):
    wid = lax.axis_index("c") * _NS + lax.axis_index("s")
    pltpu.sync_copy(labels_hbm, labels_v)
    pltpu.sync_copy(ptr_hbm, ptr_v)
    lane = lax.iota(jnp.int32, _L)

    for k in range(_CPAD // _NW):
        c = wid + _NW * k

        @pl.when(c < _NCLASS)
        def _process(c=c):
            cp = pltpu.make_async_copy(
                queue_hbm.at[pl.ds(c * _QSIZE, _QSIZE)],
                out_hbm.at[pl.ds(c * _QSIZE, _QSIZE)],
                copy_sem,
            )
            cp.start()

            def scan_step(i, cnt):
                lbl = labels_v[pl.ds(i * _L, _L)]
                msk = lbl == c
                inc = plsc.cumsum(msk.astype(jnp.int32))
                plsc.store_scatter(match_v, [cnt + inc - 1], lane + i * _L,
                                   mask=msk)
                return cnt + jnp.max(inc)

            cnt = lax.fori_loop(0, _BATCH // _L, scan_step, 0)

            pv = ptr_v[pl.ds((c // _L) * _L, _L)]
            ptr_c = jnp.sum(jnp.where(lane == lax.rem(c, _L), pv, 0))

            newp = lax.rem(ptr_c + cnt, _QSIZE)
            ptrbuf_v[...] = jnp.broadcast_to(newp, (_L,))
            pltpu.sync_copy(ptrbuf_v, newptr_hbm.at[c])

            cp.wait()

            def chunk_step(j, _):
                r = lane + j * _L
                rc = jnp.minimum(r, cnt - 1)
                src = plsc.load_gather(match_v, [rc])
                dst = c * _QSIZE + lax.rem(ptr_c + rc, _QSIZE)
                pltpu.async_copy(emb_hbm.at[src], rows_v, row_sem).wait()
                pltpu.async_copy(rows_v, out_hbm.at[dst], row_sem).wait()
                return 0

            nchunks = lax.div(cnt + _L - 1, _L)
            lax.fori_loop(0, nchunks, chunk_step, 0)


_sc_update = pl.kernel(
    _sc_body,
    out_type=(
        jax.ShapeDtypeStruct((_NCLASS * _QSIZE, _FEATURE), jnp.float32),
        jax.ShapeDtypeStruct((_CPAD, _L), jnp.int32),
    ),
    mesh=plsc.VectorSubcoreMesh(core_axis_name="c", subcore_axis_name="s"),
    compiler_params=pltpu.CompilerParams(needs_layout_passes=False),
    scratch_types=[
        pltpu.VMEM((_BATCH,), jnp.int32),
        pltpu.VMEM((_CPAD,), jnp.int32),
        pltpu.VMEM((_QSIZE + _L,), jnp.int32),
        pltpu.VMEM((_L, _FEATURE), jnp.float32),
        pltpu.VMEM((_L,), jnp.int32),
        pltpu.SemaphoreType.DMA,
        pltpu.SemaphoreType.DMA,
    ],
)


def kernel(embeddings, labels, queue, queue_ptr):
    emb_norm = _normalize(embeddings.astype(jnp.float32))
    ptr_pad = jnp.pad(queue_ptr, (0, _CPAD - _NCLASS))
    queue2d = queue.reshape(_NCLASS * _QSIZE, _FEATURE)
    out2d, newptr_pad = _sc_update(labels, ptr_pad, queue2d, emb_norm)
    return (out2d.reshape(_NCLASS, _QSIZE, _FEATURE), newptr_pad[:_NCLASS, 0])


# speedup vs baseline: 21.6673x; 21.6673x over previous
import jax
import jax.numpy as jnp
from jax import lax
from jax.experimental import pallas as pl
from jax.experimental.pallas import tpu as pltpu
from jax.experimental.pallas import tpu_sc as plsc
from jax._src.pallas import mpmd as _mpmd

_FEATURE = 1024
_QSIZE = 256
_NCLASS = 100
_BATCH = 4096

_NC = 2
_NS = 16
_L = 16
_NW = _NC * _NS
_CPAD = 128


def _norm_body(emb_ref, out_ref):
    x = emb_ref[...]
    n = jnp.sqrt(jnp.sum(x * x, axis=1, keepdims=True))
    out_ref[...] = x / jnp.maximum(n, 1e-12)


def _normalize(embeddings):
    blk = 512
    return pl.pallas_call(
        _norm_body,
        grid=(_BATCH // blk,),
        in_specs=[pl.BlockSpec((blk, _FEATURE), lambda i: (i, 0))],
        out_specs=pl.BlockSpec((blk, _FEATURE), lambda i: (i, 0)),
        out_shape=jax.ShapeDtypeStruct((_BATCH, _FEATURE), jnp.float32),
    )(embeddings)


def _copy_body(src_ref, dst_ref):
    dst_ref[...] = src_ref[...]


def _copy(queue2d):
    blk = 1024
    nrows = _NCLASS * _QSIZE
    return pl.pallas_call(
        _copy_body,
        grid=(nrows // blk,),
        in_specs=[pl.BlockSpec((blk, _FEATURE), lambda i: (i, 0))],
        out_specs=pl.BlockSpec((blk, _FEATURE), lambda i: (i, 0)),
        out_shape=jax.ShapeDtypeStruct((nrows, _FEATURE), jnp.float32),
    )(queue2d)


def _sc_body(labels_hbm, ptr_hbm, qcopy_hbm, emb_hbm,
             out_hbm, newptr_hbm,
             labels_v, ptr_v, match_v, rows_v, ptrbuf_v, row_sem):
    del qcopy_hbm
    wid = lax.axis_index("c") * _NS + lax.axis_index("s")
    pltpu.sync_copy(labels_hbm, labels_v)
    pltpu.sync_copy(ptr_hbm, ptr_v)
    lane = lax.iota(jnp.int32, _L)

    for k in range(_CPAD // _NW):
        c = wid + _NW * k

        @pl.when(c < _NCLASS)
        def _process(c=c):
            def scan_step(i, cnt):
                lbl = labels_v[pl.ds(i * _L, _L)]
                msk = lbl == c
                inc = plsc.cumsum(msk.astype(jnp.int32))
                plsc.store_scatter(match_v, [cnt + inc - 1], lane + i * _L,
                                   mask=msk)
                return cnt + jnp.max(inc)

            cnt = lax.fori_loop(0, _BATCH // _L, scan_step, 0)

            pv = ptr_v[pl.ds((c // _L) * _L, _L)]
            ptr_c = jnp.sum(jnp.where(lane == lax.rem(c, _L), pv, 0))

            newp = lax.rem(ptr_c + cnt, _QSIZE)
            ptrbuf_v[...] = jnp.broadcast_to(newp, (_L,))
            pltpu.sync_copy(ptrbuf_v, newptr_hbm.at[c])

            def chunk_step(j, _):
                r = lane + j * _L
                rc = jnp.minimum(r, cnt - 1)
                src = plsc.load_gather(match_v, [rc])
                dst = c * _QSIZE + lax.rem(ptr_c + rc, _QSIZE)
                pltpu.async_copy(emb_hbm.at[src], rows_v, row_sem).wait()
                pltpu.async_copy(rows_v, out_hbm.at[dst], row_sem).wait()
                return 0

            nchunks = lax.div(cnt + _L - 1, _L)
            lax.fori_loop(0, nchunks, chunk_step, 0)


_sc_update = _mpmd._mpmd_map(
    [(plsc.VectorSubcoreMesh(core_axis_name="c", subcore_axis_name="s"),
      _sc_body)],
    (
        jax.ShapeDtypeStruct((_NCLASS * _QSIZE, _FEATURE), jnp.float32),
        jax.ShapeDtypeStruct((_CPAD, _L), jnp.int32),
    ),
    input_output_aliases={2: 0},
    scratch_types=[
        pltpu.VMEM((_BATCH,), jnp.int32),
        pltpu.VMEM((_CPAD,), jnp.int32),
        pltpu.VMEM((_QSIZE + _L,), jnp.int32),
        pltpu.VMEM((_L, _FEATURE), jnp.float32),
        pltpu.VMEM((_L,), jnp.int32),
        pltpu.SemaphoreType.DMA,
    ],
    compiler_params=pltpu.CompilerParams(needs_layout_passes=False),
)


def kernel(embeddings, labels, queue, queue_ptr):
    emb_norm = _normalize(embeddings.astype(jnp.float32))
    ptr_pad = jnp.pad(queue_ptr, (0, _CPAD - _NCLASS))
    queue2d = queue.reshape(_NCLASS * _QSIZE, _FEATURE)
    qcopy = _copy(queue2d)
    out2d, newptr_pad = _sc_update(labels, ptr_pad, qcopy, emb_norm)
    return (out2d.reshape(_NCLASS, _QSIZE, _FEATURE), newptr_pad[:_NCLASS, 0])
